# baseline (speedup 1.0000x reference)
import jax
import jax.numpy as jnp
from jax import lax
from jax.experimental import pallas as pl
from jax.experimental.pallas import tpu as pltpu
from jax.experimental.pallas import tpu_sc as plsc

B, T, N, D = 4, 12, 1000, 64
H, FH = 8, 8
FF = 256
E = 16000
NEG_SLOPE = 0.2
BT = B * T
NTOT = BT * N

_HP = 4
_TASKS = BT * _HP
_TPW = _TASKS // 32
_NPAD = 1024

_HIGH = lax.Precision.HIGHEST


def _mm_kernel(x_ref, wl_ref, wr_ref, xl_ref, xr_ref):
    x = x_ref[0]
    pad = jnp.zeros((D, _NPAD - N), jnp.float32)
    dn = (((0,), (1,)), ((), ()))
    ylt = lax.dot_general(wl_ref[...], x, dn,
                          preferred_element_type=jnp.float32,
                          precision=_HIGH)
    yrt = lax.dot_general(wr_ref[...], x, dn,
                          preferred_element_type=jnp.float32,
                          precision=_HIGH)
    xl_ref[0] = jnp.concatenate([ylt, pad], axis=1)
    xr_ref[0] = jnp.concatenate([yrt, pad], axis=1)


def _proj(x3, Wl, Wr):
    return pl.pallas_call(
        _mm_kernel,
        grid=(BT,),
        in_specs=[
            pl.BlockSpec((1, N, D), lambda i: (i, 0, 0)),
            pl.BlockSpec((D, D), lambda i: (0, 0)),
            pl.BlockSpec((D, D), lambda i: (0, 0)),
        ],
        out_specs=[
            pl.BlockSpec((1, D, _NPAD), lambda i: (i, 0, 0)),
            pl.BlockSpec((1, D, _NPAD), lambda i: (i, 0, 0)),
        ],
        out_shape=[
            jax.ShapeDtypeStruct((BT, D, _NPAD), jnp.float32),
            jax.ShapeDtypeStruct((BT, D, _NPAD), jnp.float32),
        ],
    )(x3, Wl, Wr)


def _sc_edge_body(xlt_hbm, xrt_hbm, ei_hbm, att_hbm, agg_hbm,
                  xl_v, xr_v, src_v, dst_v, den0, den1, agg_v, att_v):
    wid = lax.axis_index("s") * 2 + lax.axis_index("c")

    pltpu.sync_copy(ei_hbm.at[0], src_v)
    pltpu.sync_copy(ei_hbm.at[1], dst_v)
    pltpu.sync_copy(att_hbm, att_v)

    zeros = jnp.zeros((16,), jnp.float32)

    for t in range(_TPW):
        task = wid * _TPW + t
        bt = task // _HP
        hp16 = lax.rem(task, _HP) * 16

        pltpu.sync_copy(xlt_hbm.at[bt, pl.ds(hp16 * _NPAD, 16 * _NPAD)], xl_v)
        pltpu.sync_copy(xrt_hbm.at[bt, pl.ds(hp16 * _NPAD, 16 * _NPAD)], xr_v)

        @plsc.parallel_loop(0, _NPAD, 16)
        def _zero_den(j):
            sl16 = pl.ds(j, 16)
            den0[sl16] = zeros
            den1[sl16] = zeros

        @plsc.parallel_loop(0, _NPAD * 16, 16, unroll=4)
        def _zero_agg(j):
            agg_v[pl.ds(j, 16)] = zeros

        att_cols = [plsc.load_gather(att_v, [jnp.full((16,), 1, jnp.int32)
                                             * (hp16 + c)])
                    for c in range(16)]
        xl_sl = [xl_v.at[pl.ds(c * _NPAD, _NPAD)] for c in range(16)]
        xr_sl = [xr_v.at[pl.ds(c * _NPAD, _NPAD)] for c in range(16)]
        agg_sl = [agg_v.at[pl.ds(c * _NPAD, _NPAD)] for c in range(16)]

        dens = (den0, den1)

        @plsc.parallel_loop(0, E, 16, unroll=2)
        def _edges(i):
            sl = pl.ds(i, 16)
            sv = src_v[sl]
            dv = dst_v[sl]
            for h in range(2):
                acc = jnp.zeros((16,), jnp.float32)
                gls = []
                for k in range(8):
                    c = h * 8 + k
                    gl = plsc.load_gather(xl_sl[c], [sv])
                    gr = plsc.load_gather(xr_sl[c], [dv])
                    e = gl + gr
                    e = jnp.maximum(e, NEG_SLOPE * e)
                    acc = acc + att_cols[c] * e
                    gls.append(gl)
                p = jnp.exp(acc)
                plsc.addupdate_scatter(dens[h], [dv], p)
                for k in range(8):
                    plsc.addupdate_scatter(agg_sl[h * 8 + k], [dv],
                                           gls[k] * p)

        @plsc.parallel_loop(0, _NPAD, 16)
        def _norm(j):
            dsl = pl.ds(j, 16)
            r0 = 1.0 / (den0[dsl] + 1e-16)
            r1 = 1.0 / (den1[dsl] + 1e-16)
            for c in range(16):
                r = r0 if c < 8 else r1
                off = pl.ds(c * _NPAD + j, 16)
                agg_v[off] = agg_v[off] * r

        pltpu.sync_copy(agg_v, agg_hbm.at[bt, pl.ds(hp16 * _NPAD, 16 * _NPAD)])


def _sc_edge(xlt, xrt, ei, att64):
    mesh = plsc.VectorSubcoreMesh(core_axis_name="c", subcore_axis_name="s")
    f = pl.kernel(
        _sc_edge_body,
        out_type=jax.ShapeDtypeStruct((BT, D * _NPAD), jnp.float32),
        mesh=mesh,
        compiler_params=pltpu.CompilerParams(needs_layout_passes=False),
        scratch_types=[
            pltpu.VMEM((_NPAD * 16,), jnp.float32),
            pltpu.VMEM((_NPAD * 16,), jnp.float32),
            pltpu.VMEM((E,), jnp.int32),
            pltpu.VMEM((E,), jnp.int32),
            pltpu.VMEM((_NPAD,), jnp.float32),
            pltpu.VMEM((_NPAD,), jnp.float32),
            pltpu.VMEM((_NPAD * 16,), jnp.float32),
            pltpu.VMEM((D,), jnp.float32),
        ],
    )
    return f(xlt.reshape(BT, D * _NPAD), xrt.reshape(BT, D * _NPAD),
             ei, att64)


def _ln(v, g, b):
    mu = jnp.mean(v, axis=-1, keepdims=True)
    var = jnp.mean((v - mu) ** 2, axis=-1, keepdims=True)
    return (v - mu) * lax.rsqrt(var + 1e-5) * g + b


def _ffn_kernel(x_ref, agg_ref, w1_ref, b1_ref, w2_ref, b2_ref,
                g1_ref, be1_ref, g2_ref, be2_ref, bg_ref, o_ref):
    x = x_ref[0]
    agg_cm = agg_ref[0]
    eye = (jax.lax.broadcasted_iota(jnp.int32, (D, D), 0) ==
           jax.lax.broadcasted_iota(jnp.int32, (D, D), 1)
           ).astype(jnp.float32)
    dn = (((0,), (0,)), ((), ()))
    agg = lax.dot_general(agg_cm[:, :N], eye, dn,
                          preferred_element_type=jnp.float32,
                          precision=_HIGH)
    dn2 = (((1,), (0,)), ((), ()))
    out = _ln(x + agg + bg_ref[...], g1_ref[...], be1_ref[...])
    h = lax.dot_general(out, w1_ref[...], dn2,
                        preferred_element_type=jnp.float32, precision=_HIGH)
    h = jnp.maximum(h + b1_ref[...], 0.0)
    ff = lax.dot_general(h, w2_ref[...], dn2,
                         preferred_element_type=jnp.float32, precision=_HIGH)
    ff = ff + b2_ref[...]
    o_ref[0] = _ln(out + ff, g2_ref[...], be2_ref[...])


def _ffn(x3, aggt, b_gat, W1, b1, W2, b2, g1, be1, g2, be2):
    vec = lambda n: pl.BlockSpec((1, n), lambda i: (0, 0))
    return pl.pallas_call(
        _ffn_kernel,
        grid=(BT,),
        in_specs=[
            pl.BlockSpec((1, N, D), lambda i: (i, 0, 0)),
            pl.BlockSpec((1, D, _NPAD), lambda i: (i, 0, 0)),
            pl.BlockSpec((D, FF), lambda i: (0, 0)),
            vec(FF),
            pl.BlockSpec((FF, D), lambda i: (0, 0)),
            vec(D), vec(D), vec(D), vec(D), vec(D), vec(D),
        ],
        out_specs=pl.BlockSpec((1, N, D), lambda i: (i, 0, 0)),
        out_shape=jax.ShapeDtypeStruct((BT, N, D), jnp.float32),
    )(x3, aggt, W1, b1.reshape(1, FF), W2, b2.reshape(1, D),
      g1.reshape(1, D), be1.reshape(1, D), g2.reshape(1, D),
      be2.reshape(1, D), b_gat.reshape(1, D))


def kernel(x, n_node_edge_index, Wl, Wr, att, b_gat, W1, b1, W2, b2,
           g1, be1, g2, be2):
    x3 = x.reshape(BT, N, D)
    xlt, xrt = _proj(x3, Wl, Wr)
    order = jnp.argsort(n_node_edge_index[1])
    ei_sorted = n_node_edge_index[:, order]
    aggt = _sc_edge(xlt, xrt, ei_sorted, att.reshape(D))
    out = _ffn(x3, aggt.reshape(BT, D, _NPAD), b_gat, W1, b1, W2, b2,
               g1, be1, g2, be2)
    return out.reshape(B, T, N, D)


# speedup vs baseline: 2.3277x; 2.3277x over previous
import jax
import jax.numpy as jnp
from jax import lax
from jax.experimental import pallas as pl
from jax.experimental.pallas import tpu as pltpu
from jax.experimental.pallas import tpu_sc as plsc

B, T, N, D = 4, 12, 1000, 64
H, FH = 8, 8
FF = 256
E = 16000
NEG_SLOPE = 0.2
BT = B * T
NTOT = BT * N

_HP = 4
_TASKS = BT * _HP
_TPW = _TASKS // 32
_NPAD = 1024

_HIGH = lax.Precision.HIGHEST


def _mm_kernel(x_ref, wl_ref, wr_ref, xl_ref, xr_ref):
    x = x_ref[0]
    pad = jnp.zeros((D, _NPAD - N), jnp.float32)
    dn = (((0,), (1,)), ((), ()))
    ylt = lax.dot_general(wl_ref[...], x, dn,
                          preferred_element_type=jnp.float32,
                          precision=_HIGH)
    yrt = lax.dot_general(wr_ref[...], x, dn,
                          preferred_element_type=jnp.float32,
                          precision=_HIGH)
    xl_ref[0] = jnp.concatenate([ylt, pad], axis=1)
    xr_ref[0] = jnp.concatenate([yrt, pad], axis=1)


def _proj(x3, Wl, Wr):
    return pl.pallas_call(
        _mm_kernel,
        grid=(BT,),
        in_specs=[
            pl.BlockSpec((1, N, D), lambda i: (i, 0, 0)),
            pl.BlockSpec((D, D), lambda i: (0, 0)),
            pl.BlockSpec((D, D), lambda i: (0, 0)),
        ],
        out_specs=[
            pl.BlockSpec((1, D, _NPAD), lambda i: (i, 0, 0)),
            pl.BlockSpec((1, D, _NPAD), lambda i: (i, 0, 0)),
        ],
        out_shape=[
            jax.ShapeDtypeStruct((BT, D, _NPAD), jnp.float32),
            jax.ShapeDtypeStruct((BT, D, _NPAD), jnp.float32),
        ],
    )(x3, Wl, Wr)


def _sc_edge_body(xlt_hbm, xrt_hbm, ei_hbm, att_hbm, agg_hbm,
                  xl_v, xr_v, src_v, dst_v, den0, den1, agg_v, att_v):
    wid = lax.axis_index("s") * 2 + lax.axis_index("c")

    pltpu.sync_copy(ei_hbm.at[0], src_v)
    pltpu.sync_copy(ei_hbm.at[1], dst_v)
    pltpu.sync_copy(att_hbm, att_v)

    zeros = jnp.zeros((16,), jnp.float32)

    for t in range(_TPW):
        task = wid * _TPW + t
        bt = task // _HP
        hp16 = lax.rem(task, _HP) * 16

        pltpu.sync_copy(xlt_hbm.at[bt, pl.ds(hp16 * _NPAD, 16 * _NPAD)], xl_v)
        pltpu.sync_copy(xrt_hbm.at[bt, pl.ds(hp16 * _NPAD, 16 * _NPAD)], xr_v)

        @plsc.parallel_loop(0, _NPAD, 16)
        def _zero_den(j):
            sl16 = pl.ds(j, 16)
            den0[sl16] = zeros
            den1[sl16] = zeros

        @plsc.parallel_loop(0, _NPAD * 16, 16, unroll=4)
        def _zero_agg(j):
            agg_v[pl.ds(j, 16)] = zeros

        att_cols = [plsc.load_gather(att_v, [jnp.full((16,), 1, jnp.int32)
                                             * (hp16 + c)])
                    for c in range(16)]
        xl_sl = [xl_v.at[pl.ds(c * _NPAD, _NPAD)] for c in range(16)]
        xr_sl = [xr_v.at[pl.ds(c * _NPAD, _NPAD)] for c in range(16)]
        agg_sl = [agg_v.at[pl.ds(c * _NPAD, _NPAD)] for c in range(16)]

        dens = (den0, den1)

        @plsc.parallel_loop(0, E, 16, unroll=4)
        def _edges(i):
            sl = pl.ds(i, 16)
            sv = src_v[sl]
            dv = dst_v[sl]
            for h in range(2):
                acc = jnp.zeros((16,), jnp.float32)
                gls = []
                for k in range(8):
                    c = h * 8 + k
                    gl = plsc.load_gather(xl_sl[c], [sv])
                    gr = plsc.load_gather(xr_sl[c], [dv])
                    e = gl + gr
                    e = jnp.maximum(e, NEG_SLOPE * e)
                    acc = acc + att_cols[c] * e
                    gls.append(gl)
                p = jnp.exp(acc)
                plsc.addupdate_scatter(dens[h], [dv], p)
                for k in range(8):
                    plsc.addupdate_scatter(agg_sl[h * 8 + k], [dv],
                                           gls[k] * p)

        @plsc.parallel_loop(0, _NPAD, 16)
        def _norm(j):
            dsl = pl.ds(j, 16)
            r0 = 1.0 / (den0[dsl] + 1e-16)
            r1 = 1.0 / (den1[dsl] + 1e-16)
            for c in range(16):
                r = r0 if c < 8 else r1
                off = pl.ds(c * _NPAD + j, 16)
                agg_v[off] = agg_v[off] * r

        pltpu.sync_copy(agg_v, agg_hbm.at[bt, pl.ds(hp16 * _NPAD, 16 * _NPAD)])


def _sc_edge(xlt, xrt, ei, att64):
    mesh = plsc.VectorSubcoreMesh(core_axis_name="c", subcore_axis_name="s")
    f = pl.kernel(
        _sc_edge_body,
        out_type=jax.ShapeDtypeStruct((BT, D * _NPAD), jnp.float32),
        mesh=mesh,
        compiler_params=pltpu.CompilerParams(needs_layout_passes=False),
        scratch_types=[
            pltpu.VMEM((_NPAD * 16,), jnp.float32),
            pltpu.VMEM((_NPAD * 16,), jnp.float32),
            pltpu.VMEM((E,), jnp.int32),
            pltpu.VMEM((E,), jnp.int32),
            pltpu.VMEM((_NPAD,), jnp.float32),
            pltpu.VMEM((_NPAD,), jnp.float32),
            pltpu.VMEM((_NPAD * 16,), jnp.float32),
            pltpu.VMEM((D,), jnp.float32),
        ],
    )
    return f(xlt.reshape(BT, D * _NPAD), xrt.reshape(BT, D * _NPAD),
             ei, att64)


def _ln(v, g, b):
    mu = jnp.mean(v, axis=-1, keepdims=True)
    var = jnp.mean((v - mu) ** 2, axis=-1, keepdims=True)
    return (v - mu) * lax.rsqrt(var + 1e-5) * g + b


def _ffn_kernel(x_ref, agg_ref, w1_ref, b1_ref, w2_ref, b2_ref,
                g1_ref, be1_ref, g2_ref, be2_ref, bg_ref, o_ref):
    x = x_ref[0]
    agg_cm = agg_ref[0]
    eye = (jax.lax.broadcasted_iota(jnp.int32, (D, D), 0) ==
           jax.lax.broadcasted_iota(jnp.int32, (D, D), 1)
           ).astype(jnp.float32)
    dn = (((0,), (0,)), ((), ()))
    agg = lax.dot_general(agg_cm[:, :N], eye, dn,
                          preferred_element_type=jnp.float32,
                          precision=_HIGH)
    dn2 = (((1,), (0,)), ((), ()))
    out = _ln(x + agg + bg_ref[...], g1_ref[...], be1_ref[...])
    h = lax.dot_general(out, w1_ref[...], dn2,
                        preferred_element_type=jnp.float32, precision=_HIGH)
    h = jnp.maximum(h + b1_ref[...], 0.0)
    ff = lax.dot_general(h, w2_ref[...], dn2,
                         preferred_element_type=jnp.float32, precision=_HIGH)
    ff = ff + b2_ref[...]
    o_ref[0] = _ln(out + ff, g2_ref[...], be2_ref[...])


def _ffn(x3, aggt, b_gat, W1, b1, W2, b2, g1, be1, g2, be2):
    vec = lambda n: pl.BlockSpec((1, n), lambda i: (0, 0))
    return pl.pallas_call(
        _ffn_kernel,
        grid=(BT,),
        in_specs=[
            pl.BlockSpec((1, N, D), lambda i: (i, 0, 0)),
            pl.BlockSpec((1, D, _NPAD), lambda i: (i, 0, 0)),
            pl.BlockSpec((D, FF), lambda i: (0, 0)),
            vec(FF),
            pl.BlockSpec((FF, D), lambda i: (0, 0)),
            vec(D), vec(D), vec(D), vec(D), vec(D), vec(D),
        ],
        out_specs=pl.BlockSpec((1, N, D), lambda i: (i, 0, 0)),
        out_shape=jax.ShapeDtypeStruct((BT, N, D), jnp.float32),
    )(x3, aggt, W1, b1.reshape(1, FF), W2, b2.reshape(1, D),
      g1.reshape(1, D), be1.reshape(1, D), g2.reshape(1, D),
      be2.reshape(1, D), b_gat.reshape(1, D))


def kernel(x, n_node_edge_index, Wl, Wr, att, b_gat, W1, b1, W2, b2,
           g1, be1, g2, be2):
    x3 = x.reshape(BT, N, D)
    xlt, xrt = _proj(x3, Wl, Wr)
    aggt = _sc_edge(xlt, xrt, n_node_edge_index, att.reshape(D))
    out = _ffn(x3, aggt.reshape(BT, D, _NPAD), b_gat, W1, b1, W2, b2,
               g1, be1, g2, be2)
    return out.reshape(B, T, N, D)


# speedup vs baseline: 2.6160x; 1.1238x over previous
import jax
import jax.numpy as jnp
from jax import lax
from jax.experimental import pallas as pl
from jax.experimental.pallas import tpu as pltpu
from jax.experimental.pallas import tpu_sc as plsc

B, T, N, D = 4, 12, 1000, 64
H, FH = 8, 8
FF = 256
E = 16000
NEG_SLOPE = 0.2
BT = B * T
NTOT = BT * N

_HP = 4
_TASKS = BT * _HP
_TPW = _TASKS // 32
_NPAD = 1024

_HIGH = lax.Precision.HIGHEST


def _mm_kernel(x_ref, wl_ref, wr_ref, xl_ref, xr_ref):
    x = x_ref[0]
    pad = jnp.zeros((D, _NPAD - N), jnp.float32)
    dn = (((0,), (1,)), ((), ()))
    ylt = lax.dot_general(wl_ref[...], x, dn,
                          preferred_element_type=jnp.float32,
                          precision=_HIGH)
    yrt = lax.dot_general(wr_ref[...], x, dn,
                          preferred_element_type=jnp.float32,
                          precision=_HIGH)
    xl_ref[0] = jnp.concatenate([ylt, pad], axis=1)
    xr_ref[0] = jnp.concatenate([yrt, pad], axis=1)


def _proj(x3, Wl, Wr):
    return pl.pallas_call(
        _mm_kernel,
        grid=(BT,),
        in_specs=[
            pl.BlockSpec((1, N, D), lambda i: (i, 0, 0)),
            pl.BlockSpec((D, D), lambda i: (0, 0)),
            pl.BlockSpec((D, D), lambda i: (0, 0)),
        ],
        out_specs=[
            pl.BlockSpec((1, D, _NPAD), lambda i: (i, 0, 0)),
            pl.BlockSpec((1, D, _NPAD), lambda i: (i, 0, 0)),
        ],
        out_shape=[
            jax.ShapeDtypeStruct((BT, D, _NPAD), jnp.float32),
            jax.ShapeDtypeStruct((BT, D, _NPAD), jnp.float32),
        ],
    )(x3, Wl, Wr)


def _sc_edge_body(xlt_hbm, xrt_hbm, ei_hbm, att_hbm, agg_hbm,
                  xl_v, xr_v, src_v, dst_v, den0, den1, agg_v, att_v):
    wid = lax.axis_index("s") * 2 + lax.axis_index("c")

    pltpu.sync_copy(ei_hbm.at[0], src_v)
    pltpu.sync_copy(ei_hbm.at[1], dst_v)
    pltpu.sync_copy(att_hbm, att_v)

    zeros = jnp.zeros((16,), jnp.float32)

    for t in range(_TPW):
        task = wid * _TPW + t
        bt = task // _HP
        hp16 = lax.rem(task, _HP) * 16

        pltpu.sync_copy(xlt_hbm.at[bt, pl.ds(hp16 * _NPAD, 16 * _NPAD)], xl_v)
        pltpu.sync_copy(xrt_hbm.at[bt, pl.ds(hp16 * _NPAD, 16 * _NPAD)], xr_v)

        @plsc.parallel_loop(0, _NPAD, 16)
        def _zero_den(j):
            sl16 = pl.ds(j, 16)
            den0[sl16] = zeros
            den1[sl16] = zeros

        @plsc.parallel_loop(0, _NPAD * 16, 16, unroll=4)
        def _zero_agg(j):
            agg_v[pl.ds(j, 16)] = zeros

        att_cols = [plsc.load_gather(att_v, [jnp.full((16,), 1, jnp.int32)
                                             * (hp16 + c)])
                    for c in range(16)]
        xl_sl = [xl_v.at[pl.ds(c * _NPAD, _NPAD)] for c in range(16)]
        xr_sl = [xr_v.at[pl.ds(c * _NPAD, _NPAD)] for c in range(16)]
        agg_sl = [agg_v.at[pl.ds(c * _NPAD, _NPAD)] for c in range(16)]

        dens = (den0, den1)
        for h in range(2):
            att_h = att_cols[h * 8:(h + 1) * 8]
            den_h = dens[h]
            xl_h = xl_sl[h * 8:(h + 1) * 8]
            xr_h = xr_sl[h * 8:(h + 1) * 8]
            agg_h = agg_sl[h * 8:(h + 1) * 8]

            @plsc.parallel_loop(0, E, 16, unroll=2)
            def _edges(i, _att=att_h, _den=den_h, _xl=xl_h, _xr=xr_h,
                       _agg=agg_h):
                sl = pl.ds(i, 16)
                sv = src_v[sl]
                dv = dst_v[sl]
                acc = jnp.zeros((16,), jnp.float32)
                gls = []
                for k in range(8):
                    gl = plsc.load_gather(_xl[k], [sv])
                    gr = plsc.load_gather(_xr[k], [dv])
                    e = gl + gr
                    e = jnp.maximum(e, NEG_SLOPE * e)
                    acc = acc + _att[k] * e
                    gls.append(gl)
                p = jnp.exp(acc)
                plsc.addupdate_scatter(_den, [dv], p)
                for k in range(8):
                    plsc.addupdate_scatter(_agg[k], [dv], gls[k] * p)

        @plsc.parallel_loop(0, _NPAD, 16)
        def _norm(j):
            dsl = pl.ds(j, 16)
            r0 = 1.0 / (den0[dsl] + 1e-16)
            r1 = 1.0 / (den1[dsl] + 1e-16)
            for c in range(16):
                r = r0 if c < 8 else r1
                off = pl.ds(c * _NPAD + j, 16)
                agg_v[off] = agg_v[off] * r

        pltpu.sync_copy(agg_v, agg_hbm.at[bt, pl.ds(hp16 * _NPAD, 16 * _NPAD)])


def _sc_edge(xlt, xrt, ei, att64):
    mesh = plsc.VectorSubcoreMesh(core_axis_name="c", subcore_axis_name="s")
    f = pl.kernel(
        _sc_edge_body,
        out_type=jax.ShapeDtypeStruct((BT, D * _NPAD), jnp.float32),
        mesh=mesh,
        compiler_params=pltpu.CompilerParams(needs_layout_passes=False),
        scratch_types=[
            pltpu.VMEM((_NPAD * 16,), jnp.float32),
            pltpu.VMEM((_NPAD * 16,), jnp.float32),
            pltpu.VMEM((E,), jnp.int32),
            pltpu.VMEM((E,), jnp.int32),
            pltpu.VMEM((_NPAD,), jnp.float32),
            pltpu.VMEM((_NPAD,), jnp.float32),
            pltpu.VMEM((_NPAD * 16,), jnp.float32),
            pltpu.VMEM((D,), jnp.float32),
        ],
    )
    return f(xlt.reshape(BT, D * _NPAD), xrt.reshape(BT, D * _NPAD),
             ei, att64)


def _ln(v, g, b):
    mu = jnp.mean(v, axis=-1, keepdims=True)
    var = jnp.mean((v - mu) ** 2, axis=-1, keepdims=True)
    return (v - mu) * lax.rsqrt(var + 1e-5) * g + b


def _ffn_kernel(x_ref, agg_ref, w1_ref, b1_ref, w2_ref, b2_ref,
                g1_ref, be1_ref, g2_ref, be2_ref, bg_ref, o_ref):
    x = x_ref[0]
    agg_cm = agg_ref[0]
    eye = (jax.lax.broadcasted_iota(jnp.int32, (D, D), 0) ==
           jax.lax.broadcasted_iota(jnp.int32, (D, D), 1)
           ).astype(jnp.float32)
    dn = (((0,), (0,)), ((), ()))
    agg = lax.dot_general(agg_cm[:, :N], eye, dn,
                          preferred_element_type=jnp.float32,
                          precision=_HIGH)
    dn2 = (((1,), (0,)), ((), ()))
    out = _ln(x + agg + bg_ref[...], g1_ref[...], be1_ref[...])
    h = lax.dot_general(out, w1_ref[...], dn2,
                        preferred_element_type=jnp.float32, precision=_HIGH)
    h = jnp.maximum(h + b1_ref[...], 0.0)
    ff = lax.dot_general(h, w2_ref[...], dn2,
                         preferred_element_type=jnp.float32, precision=_HIGH)
    ff = ff + b2_ref[...]
    o_ref[0] = _ln(out + ff, g2_ref[...], be2_ref[...])


def _ffn(x3, aggt, b_gat, W1, b1, W2, b2, g1, be1, g2, be2):
    vec = lambda n: pl.BlockSpec((1, n), lambda i: (0, 0))
    return pl.pallas_call(
        _ffn_kernel,
        grid=(BT,),
        in_specs=[
            pl.BlockSpec((1, N, D), lambda i: (i, 0, 0)),
            pl.BlockSpec((1, D, _NPAD), lambda i: (i, 0, 0)),
            pl.BlockSpec((D, FF), lambda i: (0, 0)),
            vec(FF),
            pl.BlockSpec((FF, D), lambda i: (0, 0)),
            vec(D), vec(D), vec(D), vec(D), vec(D), vec(D),
        ],
        out_specs=pl.BlockSpec((1, N, D), lambda i: (i, 0, 0)),
        out_shape=jax.ShapeDtypeStruct((BT, N, D), jnp.float32),
    )(x3, aggt, W1, b1.reshape(1, FF), W2, b2.reshape(1, D),
      g1.reshape(1, D), be1.reshape(1, D), g2.reshape(1, D),
      be2.reshape(1, D), b_gat.reshape(1, D))


def kernel(x, n_node_edge_index, Wl, Wr, att, b_gat, W1, b1, W2, b2,
           g1, be1, g2, be2):
    x3 = x.reshape(BT, N, D)
    xlt, xrt = _proj(x3, Wl, Wr)
    aggt = _sc_edge(xlt, xrt, n_node_edge_index, att.reshape(D))
    out = _ffn(x3, aggt.reshape(BT, D, _NPAD), b_gat, W1, b1, W2, b2,
               g1, be1, g2, be2)
    return out.reshape(B, T, N, D)


# speedup vs baseline: 3.2922x; 1.2585x over previous
import jax
import jax.numpy as jnp
from jax import lax
from jax.experimental import pallas as pl
from jax.experimental.pallas import tpu as pltpu
from jax.experimental.pallas import tpu_sc as plsc

B, T, N, D = 4, 12, 1000, 64
H, FH = 8, 8
FF = 256
E = 16000
NEG_SLOPE = 0.2
BT = B * T
NTOT = BT * N

_HP = 4
_TASKS = BT * _HP
_TPW = _TASKS // 32
_NPAD = 1024

_HIGH = lax.Precision.HIGHEST


def _mm_kernel(x_ref, wl_ref, wr_ref, xl_ref, xr_ref):
    x = x_ref[0]
    pad = jnp.zeros((D, _NPAD - N), jnp.float32)
    dn = (((0,), (1,)), ((), ()))
    ylt = lax.dot_general(wl_ref[...], x, dn,
                          preferred_element_type=jnp.float32,
                          precision=lax.Precision.DEFAULT)
    yrt = lax.dot_general(wr_ref[...], x, dn,
                          preferred_element_type=jnp.float32,
                          precision=lax.Precision.DEFAULT)
    xl_ref[0] = jnp.concatenate([ylt, pad], axis=1)
    xr_ref[0] = jnp.concatenate([yrt, pad], axis=1)


def _proj(x3, Wl, Wr):
    return pl.pallas_call(
        _mm_kernel,
        grid=(BT,),
        in_specs=[
            pl.BlockSpec((1, N, D), lambda i: (i, 0, 0)),
            pl.BlockSpec((D, D), lambda i: (0, 0)),
            pl.BlockSpec((D, D), lambda i: (0, 0)),
        ],
        out_specs=[
            pl.BlockSpec((1, D, _NPAD), lambda i: (i, 0, 0)),
            pl.BlockSpec((1, D, _NPAD), lambda i: (i, 0, 0)),
        ],
        out_shape=[
            jax.ShapeDtypeStruct((BT, D, _NPAD), jnp.float32),
            jax.ShapeDtypeStruct((BT, D, _NPAD), jnp.float32),
        ],
    )(x3, Wl, Wr)


def _sc_edge_body(xlt_hbm, xrt_hbm, ei_hbm, att_hbm, agg_hbm,
                  xl_v, xr_v, src_v, dst_v, den0, den1, agg_v, att_v):
    wid = lax.axis_index("s") * 2 + lax.axis_index("c")

    pltpu.sync_copy(ei_hbm.at[0], src_v)
    pltpu.sync_copy(ei_hbm.at[1], dst_v)
    pltpu.sync_copy(att_hbm, att_v)

    zeros = jnp.zeros((16,), jnp.float32)

    for t in range(_TPW):
        task = wid * _TPW + t
        bt = task // _HP
        hp16 = lax.rem(task, _HP) * 16

        pltpu.sync_copy(xlt_hbm.at[bt, pl.ds(hp16 * _NPAD, 16 * _NPAD)], xl_v)
        pltpu.sync_copy(xrt_hbm.at[bt, pl.ds(hp16 * _NPAD, 16 * _NPAD)], xr_v)

        @plsc.parallel_loop(0, _NPAD, 16)
        def _zero_den(j):
            sl16 = pl.ds(j, 16)
            den0[sl16] = zeros
            den1[sl16] = zeros

        @plsc.parallel_loop(0, _NPAD * 16, 16, unroll=4)
        def _zero_agg(j):
            agg_v[pl.ds(j, 16)] = zeros

        att_cols = [plsc.load_gather(att_v, [jnp.full((16,), 1, jnp.int32)
                                             * (hp16 + c)])
                    for c in range(16)]
        xl_sl = [xl_v.at[pl.ds(c * _NPAD, _NPAD)] for c in range(16)]
        xr_sl = [xr_v.at[pl.ds(c * _NPAD, _NPAD)] for c in range(16)]
        agg_sl = [agg_v.at[pl.ds(c * _NPAD, _NPAD)] for c in range(16)]

        dens = (den0, den1)
        for h in range(2):
            att_h = att_cols[h * 8:(h + 1) * 8]
            den_h = dens[h]
            xl_h = xl_sl[h * 8:(h + 1) * 8]
            xr_h = xr_sl[h * 8:(h + 1) * 8]
            agg_h = agg_sl[h * 8:(h + 1) * 8]

            @plsc.parallel_loop(0, E, 16, unroll=2)
            def _edges(i, _att=att_h, _den=den_h, _xl=xl_h, _xr=xr_h,
                       _agg=agg_h):
                sl = pl.ds(i, 16)
                sv = src_v[sl]
                dv = dst_v[sl]
                acc = jnp.zeros((16,), jnp.float32)
                gls = []
                for k in range(8):
                    gl = plsc.load_gather(_xl[k], [sv])
                    gr = plsc.load_gather(_xr[k], [dv])
                    e = gl + gr
                    e = jnp.maximum(e, NEG_SLOPE * e)
                    acc = acc + _att[k] * e
                    gls.append(gl)
                p = jnp.exp(acc)
                plsc.addupdate_scatter(_den, [dv], p)
                for k in range(8):
                    plsc.addupdate_scatter(_agg[k], [dv], gls[k] * p)

        @plsc.parallel_loop(0, _NPAD, 16)
        def _norm(j):
            dsl = pl.ds(j, 16)
            r0 = 1.0 / (den0[dsl] + 1e-16)
            r1 = 1.0 / (den1[dsl] + 1e-16)
            for c in range(16):
                r = r0 if c < 8 else r1
                off = pl.ds(c * _NPAD + j, 16)
                agg_v[off] = agg_v[off] * r

        pltpu.sync_copy(agg_v, agg_hbm.at[bt, pl.ds(hp16 * _NPAD, 16 * _NPAD)])


def _sc_edge(xlt, xrt, ei, att64):
    mesh = plsc.VectorSubcoreMesh(core_axis_name="c", subcore_axis_name="s")
    f = pl.kernel(
        _sc_edge_body,
        out_type=jax.ShapeDtypeStruct((BT, D * _NPAD), jnp.float32),
        mesh=mesh,
        compiler_params=pltpu.CompilerParams(needs_layout_passes=False),
        scratch_types=[
            pltpu.VMEM((_NPAD * 16,), jnp.float32),
            pltpu.VMEM((_NPAD * 16,), jnp.float32),
            pltpu.VMEM((E,), jnp.int32),
            pltpu.VMEM((E,), jnp.int32),
            pltpu.VMEM((_NPAD,), jnp.float32),
            pltpu.VMEM((_NPAD,), jnp.float32),
            pltpu.VMEM((_NPAD * 16,), jnp.float32),
            pltpu.VMEM((D,), jnp.float32),
        ],
    )
    return f(xlt.reshape(BT, D * _NPAD), xrt.reshape(BT, D * _NPAD),
             ei, att64)


def _ln(v, g, b):
    mu = jnp.mean(v, axis=-1, keepdims=True)
    var = jnp.mean((v - mu) ** 2, axis=-1, keepdims=True)
    return (v - mu) * lax.rsqrt(var + 1e-5) * g + b


def _ffn_kernel(x_ref, agg_ref, w1_ref, b1_ref, w2_ref, b2_ref,
                g1_ref, be1_ref, g2_ref, be2_ref, bg_ref, o_ref):
    x = x_ref[0]
    b1v = b1_ref[...][None, :]
    b2v = b2_ref[...][None, :]
    g1v = g1_ref[...][None, :]
    be1v = be1_ref[...][None, :]
    g2v = g2_ref[...][None, :]
    be2v = be2_ref[...][None, :]
    bgv = bg_ref[...][None, :]
    agg_cm = agg_ref[0]
    eye = (jax.lax.broadcasted_iota(jnp.int32, (D, D), 0) ==
           jax.lax.broadcasted_iota(jnp.int32, (D, D), 1)
           ).astype(jnp.float32)
    dn = (((0,), (0,)), ((), ()))
    agg = lax.dot_general(agg_cm[:, :N], eye, dn,
                          preferred_element_type=jnp.float32,
                          precision=_HIGH)
    dn2 = (((1,), (0,)), ((), ()))
    out = _ln(x + agg + bgv, g1v, be1v)
    h = lax.dot_general(out, w1_ref[...], dn2,
                        preferred_element_type=jnp.float32,
                        precision=lax.Precision.DEFAULT)
    h = jnp.maximum(h + b1v, 0.0)
    ff = lax.dot_general(h, w2_ref[...], dn2,
                         preferred_element_type=jnp.float32,
                         precision=lax.Precision.DEFAULT)
    ff = ff + b2v
    o_ref[0] = _ln(out + ff, g2v, be2v)


def _ffn(x3, aggt, b_gat, W1, b1, W2, b2, g1, be1, g2, be2):
    vec = lambda n: pl.BlockSpec((n,), lambda i: (0,))
    return pl.pallas_call(
        _ffn_kernel,
        grid=(BT,),
        in_specs=[
            pl.BlockSpec((1, N, D), lambda i: (i, 0, 0)),
            pl.BlockSpec((1, D, _NPAD), lambda i: (i, 0, 0)),
            pl.BlockSpec((D, FF), lambda i: (0, 0)),
            vec(FF),
            pl.BlockSpec((FF, D), lambda i: (0, 0)),
            vec(D), vec(D), vec(D), vec(D), vec(D), vec(D),
        ],
        out_specs=pl.BlockSpec((1, N, D), lambda i: (i, 0, 0)),
        out_shape=jax.ShapeDtypeStruct((BT, N, D), jnp.float32),
    )(x3, aggt, W1, b1, W2, b2, g1, be1, g2, be2, b_gat)


def kernel(x, n_node_edge_index, Wl, Wr, att, b_gat, W1, b1, W2, b2,
           g1, be1, g2, be2):
    x3 = x.reshape(BT, N, D)
    xlt, xrt = _proj(x3, Wl, Wr)
    aggt = _sc_edge(xlt, xrt, n_node_edge_index, att.reshape(D))
    out = _ffn(x3, aggt.reshape(BT, D, _NPAD), b_gat, W1, b1, W2, b2,
               g1, be1, g2, be2)
    return out.reshape(B, T, N, D)
